# baseline (speedup 1.0000x reference)
import functools
import math

import jax
import jax.numpy as jnp
from jax import lax
from jax.experimental import pallas as pl
from jax.experimental.pallas import tpu as pltpu
from jax.experimental.pallas import tpu_sc as plsc

D = 512
H = 8
DH = D // H
T = 512
B = 8
NEG = -1e9

SC_NC = 2
SC_NS = 16
SC_NW = SC_NC * SC_NS
ROWS = B * T
ROWS_PER_W = ROWS // SC_NW


def _ln(x, g, b):
    m = x.mean(-1, keepdims=True)
    v = ((x - m) ** 2).mean(-1, keepdims=True)
    return (x - m) / jnp.sqrt(v + 1e-5) * g + b


def _gelu_exact(x):
    return 0.5 * x * (1.0 + jax.lax.erf(x * (1.0 / math.sqrt(2.0))))


def _sc_gather(table, idx):
    mesh = plsc.VectorSubcoreMesh(core_axis_name="c", subcore_axis_name="s")

    @functools.partial(
        pl.kernel, mesh=mesh,
        out_type=jax.ShapeDtypeStruct((ROWS, D), jnp.float32),
        scratch_types=[
            pltpu.VMEM((ROWS_PER_W,), jnp.int32),
            pltpu.VMEM((ROWS_PER_W, D), jnp.float32),
            pltpu.SemaphoreType.DMA,
        ],
    )
    def k(table_hbm, idx_hbm, out_hbm, idx_v, rows_v, sem):
        wid = lax.axis_index("s") * SC_NC + lax.axis_index("c")
        base = wid * ROWS_PER_W
        pltpu.sync_copy(idx_hbm.at[pl.ds(base, ROWS_PER_W)], idx_v)
        pltpu.async_copy(table_hbm.at[idx_v], rows_v, sem).wait()
        pltpu.sync_copy(rows_v, out_hbm.at[pl.ds(base, ROWS_PER_W)])

    return k(table, idx)


def _scale_kernel(x_ref, mask_ref, wq_ref, wk_ref, wv_ref, bq_ref, bk_ref,
                  bv_ref, wo_ref, bo_ref, g1_ref, b1_ref, w1_ref, bf1_ref,
                  w2_ref, bf2_ref, g2_ref, b2_ref, out_ref):
    xb = x_ref[0]
    mask_row = mask_ref[0]
    scale = 1.0 / math.sqrt(DH)
    dot = functools.partial(jnp.dot, preferred_element_type=jnp.float32)
    acc = jnp.zeros((T, D), dtype=jnp.float32)
    for h in range(H):
        q = dot(xb, wq_ref[h]) + bq_ref[h]
        k = dot(xb, wk_ref[h]) + bk_ref[h]
        v = dot(xb, wv_ref[h]) + bv_ref[h]
        logits = jax.lax.dot_general(
            q, k, (((1,), (1,)), ((), ())),
            preferred_element_type=jnp.float32) * scale
        logits = jnp.where(mask_row > 0.5, NEG, logits)
        m = jnp.max(logits, axis=-1, keepdims=True)
        e = jnp.exp(logits - m)
        denom = jnp.sum(e, axis=-1, keepdims=True)
        o = dot(e, v) / denom
        acc = acc + dot(o, wo_ref[h])
    att = acc + bo_ref[0]
    x1 = _ln(xb + att, g1_ref[0], b1_ref[0])
    f = dot(x1, w1_ref[...]) + bf1_ref[0]
    f = _gelu_exact(f)
    f = dot(f, w2_ref[...]) + bf2_ref[0]
    out_ref[0] = _ln(x1 + f, g2_ref[0], b2_ref[0])


def _scale_block(x, p, mask):
    wq, wk, wv = jnp.split(p['Wqkv'], 3, axis=1)
    def heads(w):
        return w.reshape(D, H, DH).transpose(1, 0, 2)
    wqh, wkh, wvh = heads(wq), heads(wk), heads(wv)
    bq, bk, bv = jnp.split(p['bqkv'], 3)
    bqh = bq.reshape(H, 1, DH)
    bkh = bk.reshape(H, 1, DH)
    bvh = bv.reshape(H, 1, DH)
    woh = p['Wo'].reshape(H, DH, D)
    row = lambda a: a.reshape(1, -1)
    const = lambda shape: pl.BlockSpec(shape, lambda b: (0,) * len(shape))
    return pl.pallas_call(
        _scale_kernel,
        grid=(B,),
        in_specs=[
            pl.BlockSpec((1, T, D), lambda b: (b, 0, 0)),
            pl.BlockSpec((1, 1, T), lambda b: (b, 0, 0)),
            const((H, D, DH)), const((H, D, DH)), const((H, D, DH)),
            const((H, 1, DH)), const((H, 1, DH)), const((H, 1, DH)),
            const((H, DH, D)), const((1, D)),
            const((1, D)), const((1, D)),
            const((D, 4 * D)), const((1, 4 * D)),
            const((4 * D, D)), const((1, D)),
            const((1, D)), const((1, D)),
        ],
        out_specs=pl.BlockSpec((1, T, D), lambda b: (b, 0, 0)),
        out_shape=jax.ShapeDtypeStruct((B, T, D), jnp.float32),
    )(x, mask, wqh, wkh, wvh, bqh, bkh, bvh, woh, row(p['bo']),
      row(p['g1']), row(p['b1']), p['W1'], row(p['bf1']), p['W2'],
      row(p['bf2']), row(p['g2']), row(p['b2']))


def _bind_agg_kernel(x_ref, wkb_ref, bkb_ref, wqb_ref, bqb_ref, w1_ref, b1_ref,
                     g_ref, bn_ref, w2_ref, b2_ref, ch_ref, pad_ref, seg_ref):
    xb = x_ref[0]
    dot = functools.partial(jnp.dot, preferred_element_type=jnp.float32)
    keys = dot(xb, wkb_ref[...]) + bkb_ref[0]
    qs = dot(xb, wqb_ref[...]) + bqb_ref[0]
    keys_prev = pltpu.roll(keys, 1, 0)
    z = jnp.sum(keys_prev * qs, axis=1, keepdims=True)
    rows = jax.lax.broadcasted_iota(jnp.int32, (T, T), 0)
    cols = jax.lax.broadcasted_iota(jnp.int32, (T, T), 1)
    row_idx = jax.lax.broadcasted_iota(jnp.int32, (T, 1), 0)
    bmask = jnp.where((z > 0.0) & (row_idx > 0), 1.0, 0.0)
    starts = 1.0 - bmask
    lower = (rows <= cols).astype(jnp.float32)
    seg = jnp.sum(starts * lower, axis=0, keepdims=True) - 1.0
    seg_i = seg.astype(jnp.int32)
    s_mat = (rows == seg_i).astype(jnp.float32)
    counts = jnp.sum(s_mat, axis=1, keepdims=True)
    sums = dot(s_mat, xb)
    means = sums / jnp.maximum(counts, 1.0)
    h = _ln(dot(means, w1_ref[...]) + b1_ref[0], g_ref[0], bn_ref[0])
    h = jnp.maximum(h, 0.0)
    out = dot(h, w2_ref[...]) + b2_ref[0]
    valid = (counts > 0.0).astype(jnp.float32)
    ch_ref[0] = out * valid
    pad_ref[0] = 1.0 - valid.reshape(1, T)
    seg_ref[0] = seg_i + pl.program_id(0) * T


def _bind_agg(x, bp, ap):
    row = lambda a: a.reshape(1, -1)
    const = lambda shape: pl.BlockSpec(shape, lambda b: (0,) * len(shape))
    return pl.pallas_call(
        _bind_agg_kernel,
        grid=(B,),
        in_specs=[
            pl.BlockSpec((1, T, D), lambda b: (b, 0, 0)),
            const((D, D // 2)), const((1, D // 2)),
            const((D, D // 2)), const((1, D // 2)),
            const((D, D)), const((1, D)),
            const((1, D)), const((1, D)),
            const((D, D)), const((1, D)),
        ],
        out_specs=[
            pl.BlockSpec((1, T, D), lambda b: (b, 0, 0)),
            pl.BlockSpec((1, 1, T), lambda b: (b, 0, 0)),
            pl.BlockSpec((1, 1, T), lambda b: (b, 0, 0)),
        ],
        out_shape=[
            jax.ShapeDtypeStruct((B, T, D), jnp.float32),
            jax.ShapeDtypeStruct((B, 1, T), jnp.float32),
            jax.ShapeDtypeStruct((B, 1, T), jnp.int32),
        ],
    )(x, bp['Wk'], row(bp['bk']), bp['Wq'], row(bp['bq']),
      ap['W1'], row(ap['b1']), row(ap['g']), row(ap['bn']),
      ap['W2'], row(ap['b2']))


def _infl_kernel(h_ref, w_ref, b_ref, out_ref):
    out_ref[0] = jnp.dot(h_ref[0], w_ref[...],
                         preferred_element_type=jnp.float32) + b_ref[0]


def _gate_infl_kernel(repl_ref, g_ref, w_ref, b_ref, out_ref):
    x = repl_ref[0] * (1.0 + jax.nn.sigmoid(g_ref[0]))
    out_ref[0] = jnp.dot(x, w_ref[...],
                         preferred_element_type=jnp.float32) + b_ref[0]


def _gate_kernel(repl_ref, g_ref, out_ref):
    out_ref[0] = repl_ref[0] * (1.0 + jax.nn.sigmoid(g_ref[0]))


def _bdt():
    return pl.BlockSpec((1, T, D), lambda b: (b, 0, 0))


def _infl(higher, p):
    const = lambda a: pl.BlockSpec(a.shape, lambda b: (0,) * a.ndim)
    bb = p['b'].reshape(1, -1)
    return pl.pallas_call(
        _infl_kernel, grid=(B,),
        in_specs=[_bdt(), const(p['W']), const(bb)],
        out_specs=_bdt(),
        out_shape=jax.ShapeDtypeStruct((B, T, D), jnp.float32),
    )(higher, p['W'], bb)


def _gate_infl(repl, g, p):
    const = lambda a: pl.BlockSpec(a.shape, lambda b: (0,) * a.ndim)
    bb = p['b'].reshape(1, -1)
    return pl.pallas_call(
        _gate_infl_kernel, grid=(B,),
        in_specs=[_bdt(), _bdt(), const(p['W']), const(bb)],
        out_specs=_bdt(),
        out_shape=jax.ShapeDtypeStruct((B, T, D), jnp.float32),
    )(repl, g, p['W'], bb)


def _gate(repl, g):
    return pl.pallas_call(
        _gate_kernel, grid=(B,),
        in_specs=[_bdt(), _bdt()],
        out_specs=_bdt(),
        out_shape=jax.ShapeDtypeStruct((B, T, D), jnp.float32),
    )(repl, g)


def kernel(x, params):
    scales = params['scales']
    binds = params['binds']
    aggs = params['aggs']
    downs = params['downs']
    zero_mask = jnp.zeros((B, 1, T), dtype=jnp.float32)
    rep0 = _scale_block(x, scales[0], zero_mask)
    ch0, pad0, seg0 = _bind_agg(rep0, binds[0], aggs[0])
    rep1 = _scale_block(ch0, scales[1], pad0)
    ch1, pad1, seg1 = _bind_agg(rep1, binds[1], aggs[1])
    rep2 = _scale_block(ch1, scales[2], pad1)
    infl1 = _infl(rep2, downs[1])
    g1 = _sc_gather(infl1.reshape(ROWS, D),
                    seg1.reshape(ROWS)).reshape(B, T, D)
    infl0 = _gate_infl(rep1, g1, downs[0])
    g0 = _sc_gather(infl0.reshape(ROWS, D),
                    seg0.reshape(ROWS)).reshape(B, T, D)
    return _gate(rep0, g0)


# speedup vs baseline: 1.0466x; 1.0466x over previous
import functools
import math

import jax
import jax.numpy as jnp
from jax import lax
from jax.experimental import pallas as pl
from jax.experimental.pallas import tpu as pltpu
from jax.experimental.pallas import tpu_sc as plsc

D = 512
H = 8
DH = D // H
T = 512
B = 8
NEG = -1e9

SC_NC = 2
SC_NS = 16
SC_NW = SC_NC * SC_NS
ROWS = B * T
ROWS_PER_W = ROWS // SC_NW


def _ln(x, g, b):
    m = x.mean(-1, keepdims=True)
    v = ((x - m) ** 2).mean(-1, keepdims=True)
    return (x - m) / jnp.sqrt(v + 1e-5) * g + b


def _gelu_exact(x):
    return 0.5 * x * (1.0 + jax.lax.erf(x * (1.0 / math.sqrt(2.0))))


def _sc_gather(table, idx):
    mesh = plsc.VectorSubcoreMesh(core_axis_name="c", subcore_axis_name="s")

    @functools.partial(
        pl.kernel, mesh=mesh,
        out_type=jax.ShapeDtypeStruct((ROWS, D), jnp.float32),
        scratch_types=[
            pltpu.VMEM((ROWS_PER_W,), jnp.int32),
            pltpu.VMEM((ROWS_PER_W, D), jnp.float32),
            pltpu.SemaphoreType.DMA,
        ],
    )
    def k(table_hbm, idx_hbm, out_hbm, idx_v, rows_v, sem):
        wid = lax.axis_index("s") * SC_NC + lax.axis_index("c")
        base = wid * ROWS_PER_W
        pltpu.sync_copy(idx_hbm.at[pl.ds(base, ROWS_PER_W)], idx_v)
        pltpu.async_copy(table_hbm.at[idx_v], rows_v, sem).wait()
        pltpu.sync_copy(rows_v, out_hbm.at[pl.ds(base, ROWS_PER_W)])

    return k(table, idx)


def _attn_ffn(xb, mask_row, wq_ref, wk_ref, wv_ref, bq_ref, bk_ref, bv_ref,
              wo_ref, bo_ref, g1_ref, b1_ref, w1_ref, bf1_ref, w2_ref,
              bf2_ref, g2_ref, b2_ref):
    scale = 1.0 / math.sqrt(DH)
    dot = functools.partial(jnp.dot, preferred_element_type=jnp.float32)
    acc = jnp.zeros((T, D), dtype=jnp.float32)
    for h in range(H):
        q = dot(xb, wq_ref[h]) + bq_ref[h]
        k = dot(xb, wk_ref[h]) + bk_ref[h]
        v = dot(xb, wv_ref[h]) + bv_ref[h]
        logits = jax.lax.dot_general(
            q, k, (((1,), (1,)), ((), ())),
            preferred_element_type=jnp.float32) * scale
        if mask_row is not None:
            logits = jnp.where(mask_row > 0.5, NEG, logits)
        m = jnp.max(logits, axis=-1, keepdims=True)
        e = jnp.exp(logits - m)
        denom = jnp.sum(e, axis=-1, keepdims=True)
        o = dot(e, v) / denom
        acc = acc + dot(o, wo_ref[h])
    att = acc + bo_ref[0]
    x1 = _ln(xb + att, g1_ref[0], b1_ref[0])
    f = dot(x1, w1_ref[...]) + bf1_ref[0]
    f = _gelu_exact(f)
    f = dot(f, w2_ref[...]) + bf2_ref[0]
    return _ln(x1 + f, g2_ref[0], b2_ref[0])


def _bind_agg_body(xb, wkb_ref, bkb_ref, wqb_ref, bqb_ref, wa1_ref, ba1_ref,
                   ga_ref, bna_ref, wa2_ref, ba2_ref):
    dot = functools.partial(jnp.dot, preferred_element_type=jnp.float32)
    keys = dot(xb, wkb_ref[...]) + bkb_ref[0]
    qs = dot(xb, wqb_ref[...]) + bqb_ref[0]
    keys_prev = pltpu.roll(keys, 1, 0)
    z = jnp.sum(keys_prev * qs, axis=1, keepdims=True)
    rows = jax.lax.broadcasted_iota(jnp.int32, (T, T), 0)
    cols = jax.lax.broadcasted_iota(jnp.int32, (T, T), 1)
    row_idx = jax.lax.broadcasted_iota(jnp.int32, (T, 1), 0)
    bmask = jnp.where((z > 0.0) & (row_idx > 0), 1.0, 0.0)
    starts = 1.0 - bmask
    lower = (rows <= cols).astype(jnp.float32)
    seg = jnp.sum(starts * lower, axis=0, keepdims=True) - 1.0
    seg_i = seg.astype(jnp.int32)
    s_mat = (rows == seg_i).astype(jnp.float32)
    counts = jnp.sum(s_mat, axis=1, keepdims=True)
    sums = dot(s_mat, xb)
    means = sums / jnp.maximum(counts, 1.0)
    h = _ln(dot(means, wa1_ref[...]) + ba1_ref[0], ga_ref[0], bna_ref[0])
    h = jnp.maximum(h, 0.0)
    out = dot(h, wa2_ref[...]) + ba2_ref[0]
    valid = (counts > 0.0).astype(jnp.float32)
    return out * valid, 1.0 - valid.reshape(1, T), seg_i


_N_SCALE_W = 17


def _scale_bind_kernel(x_ref, *refs):
    mask_ref = refs[0]
    scale_refs = refs[1:_N_SCALE_W]
    agg_refs = refs[_N_SCALE_W:-4]
    rep_ref, ch_ref, pad_ref, seg_ref = refs[-4:]
    mask_row = mask_ref[0] if mask_ref is not None else None
    rep = _attn_ffn(x_ref[0], mask_row, *scale_refs)
    ch, pad, seg_i = _bind_agg_body(rep, *agg_refs)
    rep_ref[0] = rep
    ch_ref[0] = ch
    pad_ref[0] = pad
    seg_ref[0] = seg_i + pl.program_id(0) * T


def _scale0_bind_kernel(x_ref, *refs):
    _scale_bind_kernel(x_ref, None, *refs)


def _scale_infl_kernel(x_ref, mask_ref, *refs):
    scale_refs = refs[:_N_SCALE_W - 1]
    wd_ref, bd_ref, out_ref = refs[_N_SCALE_W - 1:]
    rep = _attn_ffn(x_ref[0], mask_ref[0], *scale_refs)
    out_ref[0] = jnp.dot(rep, wd_ref[...],
                         preferred_element_type=jnp.float32) + bd_ref[0]


def _gate_infl_kernel(repl_ref, g_ref, w_ref, b_ref, out_ref):
    x = repl_ref[0] * (1.0 + jax.nn.sigmoid(g_ref[0]))
    out_ref[0] = jnp.dot(x, w_ref[...],
                         preferred_element_type=jnp.float32) + b_ref[0]


def _gate_kernel(repl_ref, g_ref, out_ref):
    out_ref[0] = repl_ref[0] * (1.0 + jax.nn.sigmoid(g_ref[0]))


def _bdt():
    return pl.BlockSpec((1, T, D), lambda b: (b, 0, 0))


def _row_spec():
    return pl.BlockSpec((1, 1, T), lambda b: (b, 0, 0))


def _const(a):
    return pl.BlockSpec(a.shape, lambda b: (0,) * a.ndim)


def _scale_weight_args(p):
    wq, wk, wv = jnp.split(p['Wqkv'], 3, axis=1)
    def heads(w):
        return w.reshape(D, H, DH).transpose(1, 0, 2)
    bq, bk, bv = jnp.split(p['bqkv'], 3)
    row = lambda a: a.reshape(1, -1)
    return [heads(wq), heads(wk), heads(wv),
            bq.reshape(H, 1, DH), bk.reshape(H, 1, DH), bv.reshape(H, 1, DH),
            p['Wo'].reshape(H, DH, D), row(p['bo']),
            row(p['g1']), row(p['b1']), p['W1'], row(p['bf1']),
            p['W2'], row(p['bf2']), row(p['g2']), row(p['b2'])]


def _agg_weight_args(bp, ap):
    row = lambda a: a.reshape(1, -1)
    return [bp['Wk'], row(bp['bk']), bp['Wq'], row(bp['bq']),
            ap['W1'], row(ap['b1']), row(ap['g']), row(ap['bn']),
            ap['W2'], row(ap['b2'])]


def _scale_bind(x, p, bp, ap, mask):
    wargs = _scale_weight_args(p) + _agg_weight_args(bp, ap)
    out_shape = [
        jax.ShapeDtypeStruct((B, T, D), jnp.float32),
        jax.ShapeDtypeStruct((B, T, D), jnp.float32),
        jax.ShapeDtypeStruct((B, 1, T), jnp.float32),
        jax.ShapeDtypeStruct((B, 1, T), jnp.int32),
    ]
    out_specs = [_bdt(), _bdt(), _row_spec(), _row_spec()]
    if mask is None:
        return pl.pallas_call(
            _scale0_bind_kernel, grid=(B,),
            in_specs=[_bdt()] + [_const(a) for a in wargs],
            out_specs=out_specs, out_shape=out_shape,
        )(x, *wargs)
    return pl.pallas_call(
        _scale_bind_kernel, grid=(B,),
        in_specs=[_bdt(), _row_spec()] + [_const(a) for a in wargs],
        out_specs=out_specs, out_shape=out_shape,
    )(x, mask, *wargs)


def _scale_infl(x, p, mask, dp):
    wargs = _scale_weight_args(p) + [dp['W'], dp['b'].reshape(1, -1)]
    return pl.pallas_call(
        _scale_infl_kernel, grid=(B,),
        in_specs=[_bdt(), _row_spec()] + [_const(a) for a in wargs],
        out_specs=_bdt(),
        out_shape=jax.ShapeDtypeStruct((B, T, D), jnp.float32),
    )(x, mask, *wargs)


def _gate_infl(repl, g, p):
    bb = p['b'].reshape(1, -1)
    return pl.pallas_call(
        _gate_infl_kernel, grid=(B,),
        in_specs=[_bdt(), _bdt(), _const(p['W']), _const(bb)],
        out_specs=_bdt(),
        out_shape=jax.ShapeDtypeStruct((B, T, D), jnp.float32),
    )(repl, g, p['W'], bb)


def _gate(repl, g):
    return pl.pallas_call(
        _gate_kernel, grid=(B,),
        in_specs=[_bdt(), _bdt()],
        out_specs=_bdt(),
        out_shape=jax.ShapeDtypeStruct((B, T, D), jnp.float32),
    )(repl, g)


def kernel(x, params):
    scales = params['scales']
    binds = params['binds']
    aggs = params['aggs']
    downs = params['downs']
    rep0, ch0, pad0, seg0 = _scale_bind(x, scales[0], binds[0], aggs[0], None)
    rep1, ch1, pad1, seg1 = _scale_bind(ch0, scales[1], binds[1], aggs[1],
                                        pad0)
    infl1 = _scale_infl(ch1, scales[2], pad1, downs[1])
    g1 = _sc_gather(infl1.reshape(ROWS, D),
                    seg1.reshape(ROWS)).reshape(B, T, D)
    infl0 = _gate_infl(rep1, g1, downs[0])
    g0 = _sc_gather(infl0.reshape(ROWS, D),
                    seg0.reshape(ROWS)).reshape(B, T, D)
    return _gate(rep0, g0)
